# baseline (speedup 1.0000x reference)
import functools

import jax
import jax.numpy as jnp
from jax import lax
from jax.experimental import pallas as pl
from jax.experimental.pallas import tpu as pltpu
from jax.experimental.pallas import tpu_sc as plsc

_BB = 16
_N = 360
_NP = 368
_NQ = 96
_BIG = 3.0e38


def _rank_kernel(x_ref, o_ref):
    xb = x_ref[...]
    s = lax.bitcast_convert_type(xb, jnp.int32)
    u = jnp.where(s >= 0, s, jnp.int32(-2147483648) - s)
    j = lax.broadcasted_iota(jnp.int32, xb.shape, 1)
    ku = lax.shift_left(lax.shift_right_arithmetic(u, 9), 9) + j
    m = ku[:, None, :] < ku[:, :, None]
    rank_f = jnp.sum(jnp.where(m, 1.0, 0.0), axis=-1)
    o_ref[...] = rank_f.astype(jnp.int32)


def _sc_knn_body(spw, x_hbm, r_hbm, out_hbm, xv, rk, sv, pv, hv):
    nc = 2
    wid = lax.axis_index("s") * nc + lax.axis_index("c")
    base = wid * spw
    pltpu.sync_copy(x_hbm.at[pl.ds(base * _N, spw * _N)], xv)
    pltpu.sync_copy(r_hbm.at[pl.ds(base * _N, spw * _N)], rk)
    iota = lax.iota(jnp.int32, 16)

    def body(i, carry_unused):
        offx = i * _N
        off = i * _NP
        for st in list(range(0, _N - 16, 16)) + [_N - 16]:
            xc = xv[pl.ds(offx + st, 16)]
            rc = rk[pl.ds(offx + st, 16)]
            plsc.store_scatter(sv, [off + rc], xc)
        carry = jnp.float32(0.0)
        for c in range(_NP // 16):
            sc_ = sv[pl.ds(off + 16 * c, 16)]
            pv[pl.ds(off + 16 * c, 16)] = plsc.cumsum(sc_) + carry
            carry = carry + jnp.sum(sc_)
        for c in range(_NQ // 16):
            p = iota + 16 * c
            qcol = jnp.minimum(p * 4, 359)
            r = plsc.load_gather(rk, [offx + qcol])
            q = plsc.load_gather(sv, [off + r])
            best_c = jnp.full((16,), _BIG, jnp.float32)
            best_s = jnp.zeros((16,), jnp.float32)
            for t in range(8):
                l = r - 7 + t
                feas = (l >= 0) & (l <= 352)
                lc = jnp.clip(l, 0, 352)
                left = plsc.load_gather(sv, [off + lc])
                right = plsc.load_gather(sv, [off + lc + 7])
                cost = jnp.maximum(q - left, right - q)
                cost = jnp.where(feas, cost, _BIG)
                pr = plsc.load_gather(pv, [off + lc + 7])
                pl_ = plsc.load_gather(pv, [off + jnp.maximum(lc - 1, 0)])
                wsum = pr - jnp.where(lc > 0, pl_, 0.0)
                take = cost < best_c
                best_c = jnp.where(take, cost, best_c)
                best_s = jnp.where(take, wsum, best_s)
            hv[pl.ds(i * _NQ + 16 * c, 16)] = best_s * 0.125
        return carry_unused

    lax.fori_loop(0, spw, body, jnp.int32(0))
    pltpu.sync_copy(hv, out_hbm.at[pl.ds(base * _NQ, spw * _NQ)])


def _mlp_kernel(h_ref, w1_ref, b1_ref, w2_ref, b2_ref, w3_ref, b3_ref,
                o_ref):
    h = h_ref[...][:, :90]
    h = jnp.maximum(
        jnp.dot(h, w1_ref[...], preferred_element_type=jnp.float32)
        + b1_ref[...], 0.0)
    h = jnp.maximum(
        jnp.dot(h, w2_ref[...], preferred_element_type=jnp.float32)
        + b2_ref[...], 0.0)
    h = jnp.maximum(
        jnp.dot(h, w3_ref[...], preferred_element_type=jnp.float32)
        + b3_ref[...], 0.0)
    o_ref[...] = jnp.tanh(h)


def _knn_half(xh):
    Bh = xh.shape[0]
    rank = pl.pallas_call(
        _rank_kernel,
        grid=(Bh // _BB,),
        in_specs=[
            pl.BlockSpec((_BB, _N), lambda i: (i, 0)),
        ],
        out_specs=pl.BlockSpec((_BB, _N), lambda i: (i, 0)),
        out_shape=jax.ShapeDtypeStruct((Bh, _N), jnp.int32),
    )(xh)

    spw = Bh // 32
    sc_knn = functools.partial(
        pl.kernel,
        mesh=plsc.VectorSubcoreMesh(core_axis_name="c", subcore_axis_name="s"),
        out_type=jax.ShapeDtypeStruct((Bh * _NQ,), jnp.float32),
        compiler_params=pltpu.CompilerParams(needs_layout_passes=False),
        scratch_types=[
            pltpu.VMEM((spw * _N,), jnp.float32),
            pltpu.VMEM((spw * _N,), jnp.int32),
            pltpu.VMEM((spw * _NP,), jnp.float32),
            pltpu.VMEM((spw * _NP,), jnp.float32),
            pltpu.VMEM((spw * _NQ,), jnp.float32),
        ],
    )(functools.partial(_sc_knn_body, spw))
    return sc_knn(xh.reshape(-1), rank.reshape(-1)).reshape(Bh, _NQ)


_NSPLIT = 4


@jax.jit
def kernel(x, W1, b1, W2, b2, W3, b3):
    B, N = x.shape
    Bh = B // _NSPLIT
    h = jnp.concatenate(
        [_knn_half(x[i * Bh : (i + 1) * Bh]) for i in range(_NSPLIT)])

    out = pl.pallas_call(
        _mlp_kernel,
        grid=(1,),
        in_specs=[
            pl.BlockSpec((B, _NQ), lambda i: (0, 0)),
            pl.BlockSpec((90, 256), lambda i: (0, 0)),
            pl.BlockSpec((1, 256), lambda i: (0, 0)),
            pl.BlockSpec((256, 128), lambda i: (0, 0)),
            pl.BlockSpec((1, 128), lambda i: (0, 0)),
            pl.BlockSpec((128, 128), lambda i: (0, 0)),
            pl.BlockSpec((1, 128), lambda i: (0, 0)),
        ],
        out_specs=pl.BlockSpec((B, 128), lambda i: (0, 0)),
        out_shape=jax.ShapeDtypeStruct((B, 128), jnp.float32),
    )(h, W1, b1.reshape(1, -1), W2, b2.reshape(1, -1), W3,
      b3.reshape(1, -1))
    return out


# speedup vs baseline: 1.3957x; 1.3957x over previous
import functools

import jax
import jax.numpy as jnp
from jax import lax
from jax.experimental import pallas as pl
from jax.experimental.pallas import tpu as pltpu
from jax.experimental.pallas import tpu_sc as plsc

_BB = 16
_N = 360
_NP = 368
_NQ = 96
_BIG = 3.0e38


def _rank_kernel(x_ref, o_ref):
    xb = x_ref[...]
    s = lax.bitcast_convert_type(xb, jnp.int32)
    u = jnp.where(s >= 0, s, jnp.int32(-2147483648) - s)
    j = lax.broadcasted_iota(jnp.int32, xb.shape, 1)
    ku = lax.shift_left(lax.shift_right_arithmetic(u, 9), 9) + j
    m = ku[:, :, None] < ku[:, None, :]
    rank_f = jnp.sum(jnp.where(m, 1.0, 0.0), axis=1)
    o_ref[...] = rank_f.astype(jnp.int32)


def _sc_knn_body(spw, x_hbm, r_hbm, out_hbm, xv, rk, sv, pv, hv):
    nc = 2
    wid = lax.axis_index("s") * nc + lax.axis_index("c")
    base = wid * spw
    pltpu.sync_copy(x_hbm.at[pl.ds(base * _N, spw * _N)], xv)
    pltpu.sync_copy(r_hbm.at[pl.ds(base * _N, spw * _N)], rk)
    iota = lax.iota(jnp.int32, 16)

    def body(i, carry_unused):
        offx = i * _N
        off = i * _NP
        for st in list(range(0, _N - 16, 16)) + [_N - 16]:
            xc = xv[pl.ds(offx + st, 16)]
            rc = rk[pl.ds(offx + st, 16)]
            plsc.store_scatter(sv, [off + rc], xc)
        carry = jnp.float32(0.0)
        for c in range(_NP // 16):
            sc_ = sv[pl.ds(off + 16 * c, 16)]
            pv[pl.ds(off + 16 * c, 16)] = plsc.cumsum(sc_) + carry
            carry = carry + jnp.sum(sc_)
        for c in range(_NQ // 16):
            p = iota + 16 * c
            qcol = jnp.minimum(p * 4, 359)
            r = plsc.load_gather(rk, [offx + qcol])
            q = plsc.load_gather(sv, [off + r])
            best_c = jnp.full((16,), _BIG, jnp.float32)
            best_s = jnp.zeros((16,), jnp.float32)
            for t in range(8):
                l = r - 7 + t
                feas = (l >= 0) & (l <= 352)
                lc = jnp.clip(l, 0, 352)
                left = plsc.load_gather(sv, [off + lc])
                right = plsc.load_gather(sv, [off + lc + 7])
                cost = jnp.maximum(q - left, right - q)
                cost = jnp.where(feas, cost, _BIG)
                pr = plsc.load_gather(pv, [off + lc + 7])
                pl_ = plsc.load_gather(pv, [off + jnp.maximum(lc - 1, 0)])
                wsum = pr - jnp.where(lc > 0, pl_, 0.0)
                take = cost < best_c
                best_c = jnp.where(take, cost, best_c)
                best_s = jnp.where(take, wsum, best_s)
            hv[pl.ds(i * _NQ + 16 * c, 16)] = best_s * 0.125
        return carry_unused

    lax.fori_loop(0, spw, body, jnp.int32(0))
    pltpu.sync_copy(hv, out_hbm.at[pl.ds(base * _NQ, spw * _NQ)])


def _mlp_kernel(h_ref, w1_ref, b1_ref, w2_ref, b2_ref, w3_ref, b3_ref,
                o_ref):
    h = h_ref[...][:, :90]
    h = jnp.maximum(
        jnp.dot(h, w1_ref[...], preferred_element_type=jnp.float32)
        + b1_ref[...], 0.0)
    h = jnp.maximum(
        jnp.dot(h, w2_ref[...], preferred_element_type=jnp.float32)
        + b2_ref[...], 0.0)
    h = jnp.maximum(
        jnp.dot(h, w3_ref[...], preferred_element_type=jnp.float32)
        + b3_ref[...], 0.0)
    o_ref[...] = jnp.tanh(h)


def _knn_half(xh):
    Bh = xh.shape[0]
    rank = pl.pallas_call(
        _rank_kernel,
        grid=(Bh // _BB,),
        in_specs=[
            pl.BlockSpec((_BB, _N), lambda i: (i, 0)),
        ],
        out_specs=pl.BlockSpec((_BB, _N), lambda i: (i, 0)),
        out_shape=jax.ShapeDtypeStruct((Bh, _N), jnp.int32),
    )(xh)

    spw = Bh // 32
    sc_knn = functools.partial(
        pl.kernel,
        mesh=plsc.VectorSubcoreMesh(core_axis_name="c", subcore_axis_name="s"),
        out_type=jax.ShapeDtypeStruct((Bh * _NQ,), jnp.float32),
        compiler_params=pltpu.CompilerParams(needs_layout_passes=False),
        scratch_types=[
            pltpu.VMEM((spw * _N,), jnp.float32),
            pltpu.VMEM((spw * _N,), jnp.int32),
            pltpu.VMEM((spw * _NP,), jnp.float32),
            pltpu.VMEM((spw * _NP,), jnp.float32),
            pltpu.VMEM((spw * _NQ,), jnp.float32),
        ],
    )(functools.partial(_sc_knn_body, spw))
    return sc_knn(xh.reshape(-1), rank.reshape(-1)).reshape(Bh, _NQ)


_NSPLIT = 2


@jax.jit
def kernel(x, W1, b1, W2, b2, W3, b3):
    B, N = x.shape
    Bh = B // _NSPLIT
    h = jnp.concatenate(
        [_knn_half(x[i * Bh : (i + 1) * Bh]) for i in range(_NSPLIT)])

    out = pl.pallas_call(
        _mlp_kernel,
        grid=(1,),
        in_specs=[
            pl.BlockSpec((B, _NQ), lambda i: (0, 0)),
            pl.BlockSpec((90, 256), lambda i: (0, 0)),
            pl.BlockSpec((1, 256), lambda i: (0, 0)),
            pl.BlockSpec((256, 128), lambda i: (0, 0)),
            pl.BlockSpec((1, 128), lambda i: (0, 0)),
            pl.BlockSpec((128, 128), lambda i: (0, 0)),
            pl.BlockSpec((1, 128), lambda i: (0, 0)),
        ],
        out_specs=pl.BlockSpec((B, 128), lambda i: (0, 0)),
        out_shape=jax.ShapeDtypeStruct((B, 128), jnp.float32),
    )(h, W1, b1.reshape(1, -1), W2, b2.reshape(1, -1), W3,
      b3.reshape(1, -1))
    return out


# speedup vs baseline: 1.4213x; 1.0184x over previous
import functools

import jax
import jax.numpy as jnp
from jax import lax
from jax.experimental import pallas as pl
from jax.experimental.pallas import tpu as pltpu
from jax.experimental.pallas import tpu_sc as plsc

_BB = 32
_N = 360
_NP = 368
_NQ = 96
_BIG = 3.0e38


def _rank_kernel(x_ref, o_ref):
    xb = x_ref[...]
    s = lax.bitcast_convert_type(xb, jnp.int32)
    u = jnp.where(s >= 0, s, jnp.int32(-2147483648) - s)
    j = lax.broadcasted_iota(jnp.int32, xb.shape, 1)
    ku = lax.shift_left(lax.shift_right_arithmetic(u, 9), 9) + j
    m = ku[:, :, None] < ku[:, None, :]
    rank_f = jnp.sum(jnp.where(m, 1.0, 0.0), axis=1)
    o_ref[...] = rank_f.astype(jnp.int32)


def _sc_knn_body(spw, x_hbm, r_hbm, out_hbm, xv, rk, sv, pv, hv):
    nc = 2
    wid = lax.axis_index("s") * nc + lax.axis_index("c")
    base = wid * spw
    pltpu.sync_copy(x_hbm.at[pl.ds(base * _N, spw * _N)], xv)
    pltpu.sync_copy(r_hbm.at[pl.ds(base * _N, spw * _N)], rk)
    iota = lax.iota(jnp.int32, 16)

    def body(i, carry_unused):
        offx = i * _N
        off = i * _NP
        for st in list(range(0, _N - 16, 16)) + [_N - 16]:
            xc = xv[pl.ds(offx + st, 16)]
            rc = rk[pl.ds(offx + st, 16)]
            plsc.store_scatter(sv, [off + rc], xc)
        carry = jnp.float32(0.0)
        for c in range(_NP // 16):
            sc_ = sv[pl.ds(off + 16 * c, 16)]
            pv[pl.ds(off + 16 * c, 16)] = plsc.cumsum(sc_) + carry
            carry = carry + jnp.sum(sc_)
        for c in range(_NQ // 16):
            p = iota + 16 * c
            qcol = jnp.minimum(p * 4, 359)
            r = plsc.load_gather(rk, [offx + qcol])
            q = plsc.load_gather(sv, [off + r])
            best_c = jnp.full((16,), _BIG, jnp.float32)
            best_s = jnp.zeros((16,), jnp.float32)
            for t in range(8):
                l = r - 7 + t
                feas = (l >= 0) & (l <= 352)
                lc = jnp.clip(l, 0, 352)
                left = plsc.load_gather(sv, [off + lc])
                right = plsc.load_gather(sv, [off + lc + 7])
                cost = jnp.maximum(q - left, right - q)
                cost = jnp.where(feas, cost, _BIG)
                pr = plsc.load_gather(pv, [off + lc + 7])
                pl_ = plsc.load_gather(pv, [off + jnp.maximum(lc - 1, 0)])
                wsum = pr - jnp.where(lc > 0, pl_, 0.0)
                take = cost < best_c
                best_c = jnp.where(take, cost, best_c)
                best_s = jnp.where(take, wsum, best_s)
            hv[pl.ds(i * _NQ + 16 * c, 16)] = best_s * 0.125
        return carry_unused

    lax.fori_loop(0, spw, body, jnp.int32(0))
    pltpu.sync_copy(hv, out_hbm.at[pl.ds(base * _NQ, spw * _NQ)])


def _mlp_kernel(h_ref, w1_ref, b1_ref, w2_ref, b2_ref, w3_ref, b3_ref,
                o_ref):
    h = h_ref[...][:, :90]
    h = jnp.maximum(
        jnp.dot(h, w1_ref[...], preferred_element_type=jnp.float32)
        + b1_ref[...], 0.0)
    h = jnp.maximum(
        jnp.dot(h, w2_ref[...], preferred_element_type=jnp.float32)
        + b2_ref[...], 0.0)
    h = jnp.maximum(
        jnp.dot(h, w3_ref[...], preferred_element_type=jnp.float32)
        + b3_ref[...], 0.0)
    o_ref[...] = jnp.tanh(h)


def _knn_half(xh):
    Bh = xh.shape[0]
    rank = pl.pallas_call(
        _rank_kernel,
        grid=(Bh // _BB,),
        in_specs=[
            pl.BlockSpec((_BB, _N), lambda i: (i, 0)),
        ],
        out_specs=pl.BlockSpec((_BB, _N), lambda i: (i, 0)),
        out_shape=jax.ShapeDtypeStruct((Bh, _N), jnp.int32),
    )(xh)

    spw = Bh // 32
    sc_knn = functools.partial(
        pl.kernel,
        mesh=plsc.VectorSubcoreMesh(core_axis_name="c", subcore_axis_name="s"),
        out_type=jax.ShapeDtypeStruct((Bh * _NQ,), jnp.float32),
        compiler_params=pltpu.CompilerParams(needs_layout_passes=False),
        scratch_types=[
            pltpu.VMEM((spw * _N,), jnp.float32),
            pltpu.VMEM((spw * _N,), jnp.int32),
            pltpu.VMEM((spw * _NP,), jnp.float32),
            pltpu.VMEM((spw * _NP,), jnp.float32),
            pltpu.VMEM((spw * _NQ,), jnp.float32),
        ],
    )(functools.partial(_sc_knn_body, spw))
    return sc_knn(xh.reshape(-1), rank.reshape(-1)).reshape(Bh, _NQ)


_NSPLIT = 2


@jax.jit
def kernel(x, W1, b1, W2, b2, W3, b3):
    B, N = x.shape
    Bh = B // _NSPLIT
    h = jnp.concatenate(
        [_knn_half(x[i * Bh : (i + 1) * Bh]) for i in range(_NSPLIT)])

    out = pl.pallas_call(
        _mlp_kernel,
        grid=(1,),
        in_specs=[
            pl.BlockSpec((B, _NQ), lambda i: (0, 0)),
            pl.BlockSpec((90, 256), lambda i: (0, 0)),
            pl.BlockSpec((1, 256), lambda i: (0, 0)),
            pl.BlockSpec((256, 128), lambda i: (0, 0)),
            pl.BlockSpec((1, 128), lambda i: (0, 0)),
            pl.BlockSpec((128, 128), lambda i: (0, 0)),
            pl.BlockSpec((1, 128), lambda i: (0, 0)),
        ],
        out_specs=pl.BlockSpec((B, 128), lambda i: (0, 0)),
        out_shape=jax.ShapeDtypeStruct((B, 128), jnp.float32),
    )(h, W1, b1.reshape(1, -1), W2, b2.reshape(1, -1), W3,
      b3.reshape(1, -1))
    return out
